# baseline (speedup 1.0000x reference)
import functools

import jax
import jax.numpy as jnp
from jax import lax
from jax.experimental import pallas as pl
from jax.experimental.pallas import tpu as pltpu
from jax.experimental.pallas import tpu_sc as plsc

N = 10000
E = 320000
G = 256
NB = 2
H = 2
C = 64
D = 128
ED = 64

NC = 2
NS = 16
LANES = 16
NW = NC * NS
CH = 128
NBT = E // CH
NBLK = -(-NBT // NW)
NGRP = CH // 16
S0 = 624
GSTRIPE = G // NS

_f32 = jnp.float32
_i32 = jnp.int32


def _mesh():
    return plsc.VectorSubcoreMesh(core_axis_name="c", subcore_axis_name="s",
                                  num_cores=NC, num_subcores=NS)


def _zero_rows(ref, nrows, width):
    zero16 = jnp.zeros((LANES,), _f32)
    per = width // LANES

    def body(i, _):
        ref[i // per, pl.ds((i % per) * LANES, LANES)] = zero16
        return 0
    lax.fori_loop(0, nrows * per, body, 0)


def _k_logits_body(kx_hbm, q_hbm, e0_hbm, e1_hbm, src_hbm, dst_hbm,
                   lg0_out, lg1_out, mx_out,
                   srcv, dstv, kxg, qg, ev0, ev1, lv0, lv1, mxa, sem1, sem2):
    c = lax.axis_index("c")
    s = lax.axis_index("s")
    wid = s * NC + c
    iota = lax.iota(_i32, LANES)

    mxa[...] = jnp.full((LANES,), -1e30, _f32)

    def chunk(k, _):
        bid = k * NW + wid

        @pl.when(bid < NBT)
        def _():
            base = bid * CH
            pltpu.sync_copy(src_hbm.at[pl.ds(base, CH)], srcv)
            pltpu.sync_copy(dst_hbm.at[pl.ds(base, CH)], dstv)
            cp1 = pltpu.async_copy(kx_hbm.at[srcv], kxg, sem1)
            cp2 = pltpu.async_copy(q_hbm.at[dstv], qg, sem2)
            pltpu.sync_copy(e0_hbm.at[pl.ds(base, CH)], ev0)
            pltpu.sync_copy(e1_hbm.at[pl.ds(base, CH)], ev1)
            cp1.wait()
            cp2.wait()

            def grp(g, _):
                rowv = g * 16 + iota

                def dot_head(ev, lo):
                    acc = jnp.zeros((LANES,), _f32)
                    for cc in range(C):
                        colv = jnp.full((LANES,), cc, _i32)
                        colq = jnp.full((LANES,), cc + lo, _i32)
                        qv = plsc.load_gather(qg, [rowv, colq])
                        kv = (plsc.load_gather(kxg, [rowv, colq])
                              + plsc.load_gather(ev, [rowv, colv]))
                        acc = acc + qv * kv
                    return acc

                l0 = dot_head(ev0, 0) * 0.125
                l1 = dot_head(ev1, C) * 0.125
                lv0[pl.ds(g * 16, 16)] = l0
                lv1[pl.ds(g * 16, 16)] = l1
                mxa[...] = jnp.maximum(mxa[...], jnp.maximum(l0, l1))
                return 0

            lax.fori_loop(0, NGRP, grp, 0)
            pltpu.sync_copy(lv0, lg0_out.at[pl.ds(base, CH)])
            pltpu.sync_copy(lv1, lg1_out.at[pl.ds(base, CH)])
        return 0

    lax.fori_loop(0, NBLK, chunk, 0)
    lv0[pl.ds(0, 16)] = mxa[...]
    pltpu.sync_copy(lv0, mx_out.at[pl.ds(wid * CH, CH)])


def _k_logits(kx, q, e0, e1, src, dst):
    f = pl.kernel(
        _k_logits_body,
        out_type=(jax.ShapeDtypeStruct((E,), _f32),
                  jax.ShapeDtypeStruct((E,), _f32),
                  jax.ShapeDtypeStruct((NW * CH,), _f32)),
        mesh=_mesh(),
        compiler_params=pltpu.CompilerParams(needs_layout_passes=False),
        scratch_types=(
            pltpu.VMEM((CH,), _i32),
            pltpu.VMEM((CH,), _i32),
            pltpu.VMEM((CH, D), _f32),
            pltpu.VMEM((CH, D), _f32),
            pltpu.VMEM((CH, C), _f32),
            pltpu.VMEM((CH, C), _f32),
            pltpu.VMEM((CH,), _f32),
            pltpu.VMEM((CH,), _f32),
            pltpu.VMEM((LANES,), _f32),
            pltpu.SemaphoreType.DMA,
            pltpu.SemaphoreType.DMA,
        ),
    )
    return f(kx, q, e0, e1, src, dst)


def _zero_shared(zb, sh, s, width):
    def zrow(k, _):
        pltpu.sync_copy(zb, sh.at[pl.ds(s * S0 + k * 16, 16)])
        return 0
    lax.fori_loop(0, 39, zrow, 0)

    @pl.when(s == 0)
    def _():
        pltpu.sync_copy(zb, sh.at[pl.ds(NS * S0, 16)])


def _copy_out_shared(sh, out, c, s):
    pltpu.sync_copy(sh.at[pl.ds(s * S0, S0)], out.at[c].at[pl.ds(s * S0, S0)])

    @pl.when(s == 0)
    def _():
        pltpu.sync_copy(sh.at[pl.ds(NS * S0, 16)],
                        out.at[c].at[pl.ds(NS * S0, 16)])


def _k_scatter_body(vx_hbm, e0_hbm, e1_hbm, lg0_hbm, lg1_hbm,
                    src_hbm, dst_hbm, mv_hbm,
                    acc_out,
                    srcv, dstv, vxg, ev, rows, lv, mbuf,
                    acc_sh, sem1):
    c = lax.axis_index("c")
    s = lax.axis_index("s")
    wid = s * NC + c
    iota = lax.iota(_i32, LANES)

    pltpu.sync_copy(mv_hbm, mbuf)
    mvec = mbuf[...]

    for h, (e_hbm, lg_hbm) in enumerate(
            ((e0_hbm, lg0_hbm), (e1_hbm, lg1_hbm))):
        _zero_rows(rows, CH, D)
        plsc.subcore_barrier()
        _zero_shared(rows.at[pl.ds(0, 16)], acc_sh, s, D)
        plsc.subcore_barrier()

        def chunk(k, _):
            bid = k * NW + wid

            @pl.when(bid < NBT)
            def _():
                base = bid * CH
                pltpu.sync_copy(src_hbm.at[pl.ds(base, CH)], srcv)
                pltpu.sync_copy(dst_hbm.at[pl.ds(base, CH)], dstv)
                cp1 = pltpu.async_copy(vx_hbm.at[srcv], vxg, sem1)
                pltpu.sync_copy(e_hbm.at[pl.ds(base, CH)], ev)
                pltpu.sync_copy(lg_hbm.at[pl.ds(base, CH)], lv)
                cp1.wait()

                def grp(g, _):
                    sh = jnp.exp(lv[pl.ds(g * 16, 16)] - mvec)
                    rowv = g * 16 + iota
                    plsc.store_scatter(
                        rows, [rowv, jnp.zeros((LANES,), _i32) + C], sh)

                    for cc in range(C):
                        colv = jnp.full((LANES,), cc, _i32)
                        colx = jnp.full((LANES,), cc + h * C, _i32)
                        vv = (plsc.load_gather(vxg, [rowv, colx])
                              + plsc.load_gather(ev, [rowv, colv]))
                        plsc.store_scatter(rows, [rowv, colv], sh * vv)
                    return 0

                lax.fori_loop(0, NGRP, grp, 0)
                pltpu.sync_copy(rows, acc_sh.at[dstv], add=True)
            return 0

        lax.fori_loop(0, NBLK, chunk, 0)
        plsc.subcore_barrier()
        _copy_out_shared(acc_sh, acc_out.at[h], c, s)


def _k_scatter(vx, e0, e1, lg0, lg1, src, dst, mv):
    f = pl.kernel(
        _k_scatter_body,
        out_type=jax.ShapeDtypeStruct((H, NC, N, D), _f32),
        mesh=_mesh(),
        compiler_params=pltpu.CompilerParams(needs_layout_passes=False),
        scratch_types=(
            pltpu.VMEM((CH,), _i32),
            pltpu.VMEM((CH,), _i32),
            pltpu.VMEM((CH, D), _f32),
            pltpu.VMEM((CH, C), _f32),
            pltpu.VMEM((CH, D), _f32),
            pltpu.VMEM((CH,), _f32),
            pltpu.VMEM((LANES,), _f32),
            pltpu.VMEM_SHARED((N, D), _f32),
            pltpu.SemaphoreType.DMA,
        ),
    )
    return f(vx, e0, e1, lg0, lg1, src, dst, mv)


def _k_segsum_body(x_hbm, src_hbm, dst_hbm, acc_out,
                   srcv, dstv, xg, zb, acc_sh, sem1):
    c = lax.axis_index("c")
    s = lax.axis_index("s")
    wid = s * NC + c

    _zero_rows(zb, 16, D)
    _zero_shared(zb, acc_sh, s, D)
    plsc.subcore_barrier()

    def chunk(k, _):
        bid = k * NW + wid

        @pl.when(bid < NBT)
        def _():
            base = bid * CH
            pltpu.sync_copy(src_hbm.at[pl.ds(base, CH)], srcv)
            pltpu.sync_copy(dst_hbm.at[pl.ds(base, CH)], dstv)
            pltpu.async_copy(x_hbm.at[srcv], xg, sem1).wait()
            pltpu.sync_copy(xg, acc_sh.at[dstv], add=True)
        return 0

    lax.fori_loop(0, NBLK, chunk, 0)
    plsc.subcore_barrier()
    _copy_out_shared(acc_sh, acc_out, c, s)


def _k_segsum(x, src, dst):
    f = pl.kernel(
        _k_segsum_body,
        out_type=jax.ShapeDtypeStruct((NC, N, D), _f32),
        mesh=_mesh(),
        compiler_params=pltpu.CompilerParams(needs_layout_passes=False),
        scratch_types=(
            pltpu.VMEM((CH,), _i32),
            pltpu.VMEM((CH,), _i32),
            pltpu.VMEM((CH, D), _f32),
            pltpu.VMEM((16, D), _f32),
            pltpu.VMEM_SHARED((N, D), _f32),
            pltpu.SemaphoreType.DMA,
        ),
    )
    return f(x, src, dst)


def _k_edgepool_body(v_hbm, seg_hbm, acc_out, segv, vv, vp, zb, acc_sh, sem1):
    c = lax.axis_index("c")
    s = lax.axis_index("s")
    wid = s * NC + c

    _zero_rows(vp, CH, D)
    _zero_rows(zb, GSTRIPE, D)
    pltpu.sync_copy(zb, acc_sh.at[pl.ds(s * GSTRIPE, GSTRIPE)])
    plsc.subcore_barrier()

    def chunk(k, _):
        bid = k * NW + wid

        @pl.when(bid < NBT)
        def _():
            base = bid * CH
            pltpu.sync_copy(seg_hbm.at[pl.ds(base, CH)], segv)
            pltpu.sync_copy(v_hbm.at[pl.ds(base, CH)], vv)

            def cprow(i, _):
                for r in range(ED // LANES):
                    vp[i, pl.ds(r * 16, 16)] = vv[i, pl.ds(r * 16, 16)]
                return 0
            lax.fori_loop(0, CH, cprow, 0)
            pltpu.sync_copy(vp, acc_sh.at[segv], add=True)
        return 0

    lax.fori_loop(0, NBLK, chunk, 0)
    plsc.subcore_barrier()
    off = s * GSTRIPE
    pltpu.sync_copy(acc_sh.at[pl.ds(off, GSTRIPE)],
                    acc_out.at[c].at[pl.ds(off, GSTRIPE)])


def _k_edgepool(vals, seg):
    f = pl.kernel(
        _k_edgepool_body,
        out_type=jax.ShapeDtypeStruct((NC, G, D), _f32),
        mesh=_mesh(),
        compiler_params=pltpu.CompilerParams(needs_layout_passes=False),
        scratch_types=(
            pltpu.VMEM((CH,), _i32),
            pltpu.VMEM((CH, ED), _f32),
            pltpu.VMEM((CH, D), _f32),
            pltpu.VMEM((GSTRIPE, D), _f32),
            pltpu.VMEM_SHARED((G, D), _f32),
            pltpu.SemaphoreType.DMA,
        ),
    )
    return f(vals, seg)


def _graph_ln(x, oh, ohT, cnt, gamma, beta):
    mean = (oh @ x.sum(-1)) / cnt
    var = (oh @ (x * x).sum(-1)) / cnt - mean * mean
    rstd = jax.lax.rsqrt(jnp.maximum(var, 0.0) + 1e-5)
    xn = (x - (ohT @ mean)[:, None]) * (ohT @ rstd)[:, None]
    return xn * gamma + beta


def _batch_softmax(sc, oh, ohT, ohb):
    m = jnp.where(ohb, sc[None, :], -jnp.inf).max(axis=1)
    m = jnp.where(jnp.isfinite(m), m, 0.0)
    e = jnp.exp(sc - ohT @ m)
    d = oh @ e
    return e / (ohT @ d + 1e-16)


_USE_SC_CONV = True


def _conv_jax(x, src, dst, e0, e1, p):
    q = (x @ p['wq'] + p['bq'])[dst].reshape(-1, H, C)
    k = (x[src] @ p['wk'] + p['bk']).reshape(-1, H, C)
    e = jnp.concatenate([e0, e1], axis=1).reshape(-1, H, C)
    k = k + e
    alpha = (q * k).sum(-1) / jnp.sqrt(float(C))
    m = jax.ops.segment_max(alpha, dst, num_segments=N)
    m = jnp.where(jnp.isfinite(m), m, 0.0)
    ee = jnp.exp(alpha - m[dst])
    dd = jax.ops.segment_sum(ee, dst, num_segments=N)
    alpha = ee / (dd[dst] + 1e-16)
    v = (x[src] @ p['wv'] + p['bv']).reshape(-1, H, C) + e
    out = jax.ops.segment_sum(alpha[..., None] * v, dst, num_segments=N).reshape(N, H * C)
    return out + x @ p['wskip'] + p['bskip']


def _conv(x, src, dst, e0, e1, p):
    if not _USE_SC_CONV:
        return _conv_jax(x, src, dst, e0, e1, p)
    q = x @ p['wq'] + p['bq']
    kx = x @ p['wk'] + p['bk']
    vx = x @ p['wv'] + p['bv']
    lg0, lg1, mx = _k_logits(kx, q, e0, e1, src, dst)
    mv = jnp.full((LANES,), jnp.max(mx), _f32)
    acc = _k_scatter(vx, e0, e1, lg0, lg1, src, dst, mv)
    a0 = acc[0, 0] + acc[0, 1]
    a1 = acc[1, 0] + acc[1, 1]
    out = jnp.concatenate([a0[:, :C] / (a0[:, C:C + 1] + 1e-16),
                           a1[:, :C] / (a1[:, C:C + 1] + 1e-16)], axis=1)
    return out + x @ p['wskip'] + p['bskip']


def _sag_score(x, src, dst, oh, ohT, ohb, p):
    agg2 = _k_segsum(x, src, dst)
    agg = agg2[0] + agg2[1]
    sc = (agg @ p['wrel'] + p['brel'] + x @ p['wroot'])[:, 0]
    return _batch_softmax(sc, oh, ohT, ohb)


def _side_block(bp, x, ea, src, dst, eb, oh, ohT, ohb, cnt):
    w1, b1 = bp['conv1']['we'], bp['conv1']['be']
    x = _conv(x, src, dst, ea @ w1[:, :C] + b1[:C], ea @ w1[:, C:] + b1[C:],
              bp['conv1'])
    x = jax.nn.elu(_graph_ln(x, oh, ohT, cnt, bp['n1g'], bp['n1b']))
    ea = jax.nn.elu(ea @ bp['wu1'] + bp['bu1'])
    w2, b2 = bp['conv2']['we'], bp['conv2']['be']
    x = _conv(x, src, dst, ea @ w2[:, :C] + b2[:C], ea @ w2[:, C:] + b2[C:],
              bp['conv2'])
    ea = ea @ bp['wu2'] + bp['bu2']
    s = _sag_score(x, src, dst, oh, ohT, ohb, bp['pool'])
    g = oh @ (x * s[:, None])
    eap = _k_edgepool(ea, eb)
    ge = jax.nn.elu((eap[0] + eap[1])[:, :ED] @ bp['wre'] + bp['bre'])
    g = g * ge
    x = jax.nn.elu(_graph_ln(x, oh, ohT, cnt, bp['n2g'], bp['n2b']))
    ea = jax.nn.elu(ea)
    return x, ea, g


def _l2n(v):
    return v / jnp.maximum(jnp.linalg.norm(v, axis=-1, keepdims=True), 1e-12)


_GB = 8


def _final_kernel(rh_ref, rt_ref, relm_ref, wk_ref, wq_ref, b_ref, a_ref, out_ref):
    rh = _l2n(rh_ref[...])
    rt = _l2n(rt_ref[...])
    keys = jax.lax.dot_general(rh, wk_ref[...], (((2,), (0,)), ((), ())))
    queries = jax.lax.dot_general(rt, wq_ref[...], (((2,), (0,)), ((), ())))
    eact = queries[:, None, :, :] + keys[:, :, None, :] + b_ref[...][0]
    att = jnp.sum(jnp.tanh(eact) * a_ref[...][0], axis=-1)
    relm = relm_ref[...]
    hr = jax.lax.dot_general(rh, relm, (((2,), (1,)), ((0,), (0,))))
    scores = jax.lax.dot_general(hr, rt, (((2,), (2,)), ((0,), (0,))))
    out_ref[...] = jnp.sum(att * scores, axis=(1, 2))[:, None]


def _final_stage(rh, rt, relm, wk_att, wq_att, b_att, a_att):
    return pl.pallas_call(
        _final_kernel,
        grid=(G // _GB,),
        in_specs=[
            pl.BlockSpec((_GB, NB, D), lambda i: (i, 0, 0)),
            pl.BlockSpec((_GB, NB, D), lambda i: (i, 0, 0)),
            pl.BlockSpec((_GB, D, D), lambda i: (i, 0, 0)),
            pl.BlockSpec((D, D // 2), lambda i: (0, 0)),
            pl.BlockSpec((D, D // 2), lambda i: (0, 0)),
            pl.BlockSpec((1, D // 2), lambda i: (0, 0)),
            pl.BlockSpec((1, D // 2), lambda i: (0, 0)),
        ],
        out_specs=pl.BlockSpec((_GB, 1), lambda i: (i, 0)),
        out_shape=jax.ShapeDtypeStruct((G, 1), jnp.float32),
    )(rh, rt, relm, wk_att, wq_att, b_att, a_att).reshape(G)


def kernel(params, h_x, h_edge_attr, t_x, t_edge_attr, h_fin, h_desc, t_fin, t_desc, h_edge_index, h_batch, h_edge_batch, t_edge_index, t_batch, t_edge_batch, rels):
    gids = jnp.arange(G, dtype=h_batch.dtype)
    h_ohb = h_batch[None, :] == gids[:, None]
    t_ohb = t_batch[None, :] == gids[:, None]
    h_oh = h_ohb.astype(_f32)
    t_oh = t_ohb.astype(_f32)
    h_ohT = h_oh.T
    t_ohT = t_oh.T
    h_cnt = jnp.maximum(h_oh.sum(1) * D, 1.0)
    t_cnt = jnp.maximum(t_oh.sum(1) * D, 1.0)

    hx = h_x @ params['w_in'] + params['b_in']
    tx = t_x @ params['w_in'] + params['b_in']
    hx = jax.nn.elu(_graph_ln(hx, h_oh, h_ohT, h_cnt, params['n0g'], params['n0b']))
    tx = jax.nn.elu(_graph_ln(tx, t_oh, t_ohT, t_cnt, params['n0g'], params['n0b']))
    hea = jax.nn.elu(h_edge_attr @ params['w_e'] + params['b_e'])
    tea = jax.nn.elu(t_edge_attr @ params['w_e'] + params['b_e'])

    hsrc, hdst = h_edge_index[0], h_edge_index[1]
    tsrc, tdst = t_edge_index[0], t_edge_index[1]

    reprs_h, reprs_t = [], []
    for bp in params['blocks']:
        hx, hea, hg = _side_block(bp, hx, hea, hsrc, hdst, h_edge_batch, h_oh, h_ohT, h_ohb, h_cnt)
        tx, tea, tg = _side_block(bp, tx, tea, tsrc, tdst, t_edge_batch, t_oh, t_ohT, t_ohb, t_cnt)
        reprs_h.append(hg)
        reprs_t.append(tg)

    hf = jax.nn.relu(h_fin @ params['w_fp'] + params['b_fp'])
    tf = jax.nn.relu(t_fin @ params['w_fp'] + params['b_fp'])
    rh = jnp.stack([_l2n(r + hf) for r in reprs_h], axis=1)
    rt = jnp.stack([_l2n(r + tf) for r in reprs_t], axis=1)
    relm = params['rel_emb'][rels].reshape(-1, D, D)
    return _final_stage(
        rh, rt, relm,
        params['wk_att'], params['wq_att'],
        params['b_att'].reshape(1, D // 2), params['a_att'].reshape(1, D // 2),
    )


# speedup vs baseline: 1.1237x; 1.1237x over previous
import functools

import jax
import jax.numpy as jnp
from jax import lax
from jax.experimental import pallas as pl
from jax.experimental.pallas import tpu as pltpu
from jax.experimental.pallas import tpu_sc as plsc

N = 10000
E = 320000
G = 256
NB = 2
H = 2
C = 64
D = 128
ED = 64

NC = 2
NS = 16
LANES = 16
NW = NC * NS
CH = 128
NBT = E // CH
NBLK = -(-NBT // NW)
NGRP = CH // 16
S0 = 624
GSTRIPE = G // NS

_f32 = jnp.float32
_i32 = jnp.int32


def _mesh():
    return plsc.VectorSubcoreMesh(core_axis_name="c", subcore_axis_name="s",
                                  num_cores=NC, num_subcores=NS)


def _zero_rows(ref, nrows, width):
    zero16 = jnp.zeros((LANES,), _f32)
    per = width // LANES

    def body(i, _):
        ref[i // per, pl.ds((i % per) * LANES, LANES)] = zero16
        return 0
    lax.fori_loop(0, nrows * per, body, 0)


def _k_logits_body(kx_hbm, q_hbm, e0_hbm, e1_hbm, src_hbm, dst_hbm,
                   lg0_out, lg1_out, mx_out,
                   srcv, dstv, kxg, qg, ev0, ev1, lv0, lv1, mxa, sem1, sem2):
    c = lax.axis_index("c")
    s = lax.axis_index("s")
    wid = s * NC + c
    iota = lax.iota(_i32, LANES)

    mxa[...] = jnp.full((LANES,), -1e30, _f32)

    def chunk(k, _):
        bid = k * NW + wid

        @pl.when(bid < NBT)
        def _():
            base = bid * CH
            ca = pltpu.async_copy(src_hbm.at[pl.ds(base, CH)], srcv, sem1)
            cb = pltpu.async_copy(dst_hbm.at[pl.ds(base, CH)], dstv, sem1)
            cc_ = pltpu.async_copy(e0_hbm.at[pl.ds(base, CH)], ev0, sem2)
            cd = pltpu.async_copy(e1_hbm.at[pl.ds(base, CH)], ev1, sem2)
            ca.wait()
            cb.wait()
            cp1 = pltpu.async_copy(kx_hbm.at[srcv], kxg, sem1)
            cp2 = pltpu.async_copy(q_hbm.at[dstv], qg, sem1)
            cc_.wait()
            cd.wait()
            cp1.wait()
            cp2.wait()

            def grp(g, _):
                rowv = g * 16 + iota

                def dot_head(ev, lo):
                    def cbody(cc, acc):
                        colv = jnp.zeros((LANES,), _i32) + cc
                        qv = plsc.load_gather(qg, [rowv, colv + lo])
                        kv = (plsc.load_gather(kxg, [rowv, colv + lo])
                              + plsc.load_gather(ev, [rowv, colv]))
                        return acc + qv * kv
                    return lax.fori_loop(0, C, cbody,
                                         jnp.zeros((LANES,), _f32))

                l0 = dot_head(ev0, 0) * 0.125
                l1 = dot_head(ev1, C) * 0.125
                lv0[pl.ds(g * 16, 16)] = l0
                lv1[pl.ds(g * 16, 16)] = l1
                mxa[...] = jnp.maximum(mxa[...], jnp.maximum(l0, l1))
                return 0

            lax.fori_loop(0, NGRP, grp, 0)
            cw1 = pltpu.async_copy(lv0, lg0_out.at[pl.ds(base, CH)], sem1)
            cw2 = pltpu.async_copy(lv1, lg1_out.at[pl.ds(base, CH)], sem1)
            cw1.wait()
            cw2.wait()
        return 0

    lax.fori_loop(0, NBLK, chunk, 0)
    lv0[pl.ds(0, 16)] = mxa[...]
    pltpu.sync_copy(lv0, mx_out.at[pl.ds(wid * CH, CH)])


def _k_logits(kx, q, e0, e1, src, dst):
    f = pl.kernel(
        _k_logits_body,
        out_type=(jax.ShapeDtypeStruct((E,), _f32),
                  jax.ShapeDtypeStruct((E,), _f32),
                  jax.ShapeDtypeStruct((NW * CH,), _f32)),
        mesh=_mesh(),
        compiler_params=pltpu.CompilerParams(needs_layout_passes=False),
        scratch_types=(
            pltpu.VMEM((CH,), _i32),
            pltpu.VMEM((CH,), _i32),
            pltpu.VMEM((CH, D), _f32),
            pltpu.VMEM((CH, D), _f32),
            pltpu.VMEM((CH, C), _f32),
            pltpu.VMEM((CH, C), _f32),
            pltpu.VMEM((CH,), _f32),
            pltpu.VMEM((CH,), _f32),
            pltpu.VMEM((LANES,), _f32),
            pltpu.SemaphoreType.DMA,
            pltpu.SemaphoreType.DMA,
        ),
    )
    return f(kx, q, e0, e1, src, dst)


def _zero_shared(zb, sh, s, width):
    def zrow(k, _):
        pltpu.sync_copy(zb, sh.at[pl.ds(s * S0 + k * 16, 16)])
        return 0
    lax.fori_loop(0, 39, zrow, 0)

    @pl.when(s == 0)
    def _():
        pltpu.sync_copy(zb, sh.at[pl.ds(NS * S0, 16)])


def _copy_out_shared(sh, out, c, s):
    pltpu.sync_copy(sh.at[pl.ds(s * S0, S0)], out.at[c].at[pl.ds(s * S0, S0)])

    @pl.when(s == 0)
    def _():
        pltpu.sync_copy(sh.at[pl.ds(NS * S0, 16)],
                        out.at[c].at[pl.ds(NS * S0, 16)])


def _k_scatter_body(vx_hbm, e0_hbm, e1_hbm, lg0_hbm, lg1_hbm,
                    src_hbm, dst_hbm, mv_hbm,
                    acc_out,
                    srcv, dstv, vxg, ev, rows, lv, mbuf,
                    acc_sh, sem1, sem2):
    c = lax.axis_index("c")
    s = lax.axis_index("s")
    wid = s * NC + c
    iota = lax.iota(_i32, LANES)

    pltpu.sync_copy(mv_hbm, mbuf)
    mvec = mbuf[...]

    for h, (e_hbm, lg_hbm) in enumerate(
            ((e0_hbm, lg0_hbm), (e1_hbm, lg1_hbm))):
        _zero_rows(rows, CH, D)
        plsc.subcore_barrier()
        _zero_shared(rows.at[pl.ds(0, 16)], acc_sh, s, D)
        plsc.subcore_barrier()

        def chunk(k, _):
            bid = k * NW + wid

            @pl.when(bid < NBT)
            def _():
                base = bid * CH
                ca = pltpu.async_copy(src_hbm.at[pl.ds(base, CH)], srcv, sem1)
                cb = pltpu.async_copy(dst_hbm.at[pl.ds(base, CH)], dstv, sem2)
                cc_ = pltpu.async_copy(e_hbm.at[pl.ds(base, CH)], ev, sem2)
                cd = pltpu.async_copy(lg_hbm.at[pl.ds(base, CH)], lv, sem2)
                ca.wait()
                cp1 = pltpu.async_copy(vx_hbm.at[srcv], vxg, sem1)
                cb.wait()
                cc_.wait()
                cd.wait()
                cp1.wait()

                def grp(g, _):
                    sh = jnp.exp(lv[pl.ds(g * 16, 16)] - mvec)
                    rowv = g * 16 + iota
                    plsc.store_scatter(
                        rows, [rowv, jnp.zeros((LANES,), _i32) + C], sh)

                    def cbody(cc, _):
                        colv = jnp.zeros((LANES,), _i32) + cc
                        vv = (plsc.load_gather(vxg, [rowv, colv + h * C])
                              + plsc.load_gather(ev, [rowv, colv]))
                        plsc.store_scatter(rows, [rowv, colv], sh * vv)
                        return 0
                    lax.fori_loop(0, C, cbody, 0)
                    return 0

                lax.fori_loop(0, NGRP, grp, 0)
                pltpu.sync_copy(rows, acc_sh.at[dstv], add=True)
            return 0

        lax.fori_loop(0, NBLK, chunk, 0)
        plsc.subcore_barrier()
        _copy_out_shared(acc_sh, acc_out.at[h], c, s)


def _k_scatter(vx, e0, e1, lg0, lg1, src, dst, mv):
    f = pl.kernel(
        _k_scatter_body,
        out_type=jax.ShapeDtypeStruct((H, NC, N, D), _f32),
        mesh=_mesh(),
        compiler_params=pltpu.CompilerParams(needs_layout_passes=False),
        scratch_types=(
            pltpu.VMEM((CH,), _i32),
            pltpu.VMEM((CH,), _i32),
            pltpu.VMEM((CH, D), _f32),
            pltpu.VMEM((CH, C), _f32),
            pltpu.VMEM((CH, D), _f32),
            pltpu.VMEM((CH,), _f32),
            pltpu.VMEM((LANES,), _f32),
            pltpu.VMEM_SHARED((N, D), _f32),
            pltpu.SemaphoreType.DMA,
            pltpu.SemaphoreType.DMA,
        ),
    )
    return f(vx, e0, e1, lg0, lg1, src, dst, mv)


def _k_segsum_body(x_hbm, src_hbm, dst_hbm, acc_out,
                   srcv, dstv, xg, zb, acc_sh, sem1):
    c = lax.axis_index("c")
    s = lax.axis_index("s")
    wid = s * NC + c

    _zero_rows(zb, 16, D)
    _zero_shared(zb, acc_sh, s, D)
    plsc.subcore_barrier()

    def chunk(k, _):
        bid = k * NW + wid

        @pl.when(bid < NBT)
        def _():
            base = bid * CH
            ca = pltpu.async_copy(src_hbm.at[pl.ds(base, CH)], srcv, sem1)
            cb = pltpu.async_copy(dst_hbm.at[pl.ds(base, CH)], dstv, sem1)
            ca.wait()
            cb.wait()
            pltpu.async_copy(x_hbm.at[srcv], xg, sem1).wait()
            pltpu.sync_copy(xg, acc_sh.at[dstv], add=True)
        return 0

    lax.fori_loop(0, NBLK, chunk, 0)
    plsc.subcore_barrier()
    _copy_out_shared(acc_sh, acc_out, c, s)


def _k_segsum(x, src, dst):
    f = pl.kernel(
        _k_segsum_body,
        out_type=jax.ShapeDtypeStruct((NC, N, D), _f32),
        mesh=_mesh(),
        compiler_params=pltpu.CompilerParams(needs_layout_passes=False),
        scratch_types=(
            pltpu.VMEM((CH,), _i32),
            pltpu.VMEM((CH,), _i32),
            pltpu.VMEM((CH, D), _f32),
            pltpu.VMEM((16, D), _f32),
            pltpu.VMEM_SHARED((N, D), _f32),
            pltpu.SemaphoreType.DMA,
        ),
    )
    return f(x, src, dst)


def _k_edgepool_body(v_hbm, seg_hbm, acc_out, segv, vv, vp, zb, acc_sh, sem1):
    c = lax.axis_index("c")
    s = lax.axis_index("s")
    wid = s * NC + c

    _zero_rows(vp, CH, D)
    _zero_rows(zb, GSTRIPE, D)
    pltpu.sync_copy(zb, acc_sh.at[pl.ds(s * GSTRIPE, GSTRIPE)])
    plsc.subcore_barrier()

    def chunk(k, _):
        bid = k * NW + wid

        @pl.when(bid < NBT)
        def _():
            base = bid * CH
            ca = pltpu.async_copy(seg_hbm.at[pl.ds(base, CH)], segv, sem1)
            cb = pltpu.async_copy(v_hbm.at[pl.ds(base, CH)], vv, sem1)
            ca.wait()
            cb.wait()

            def cprow(i, _):
                for r in range(ED // LANES):
                    vp[i, pl.ds(r * 16, 16)] = vv[i, pl.ds(r * 16, 16)]
                return 0
            lax.fori_loop(0, CH, cprow, 0)
            pltpu.sync_copy(vp, acc_sh.at[segv], add=True)
        return 0

    lax.fori_loop(0, NBLK, chunk, 0)
    plsc.subcore_barrier()
    off = s * GSTRIPE
    pltpu.sync_copy(acc_sh.at[pl.ds(off, GSTRIPE)],
                    acc_out.at[c].at[pl.ds(off, GSTRIPE)])


def _k_edgepool(vals, seg):
    f = pl.kernel(
        _k_edgepool_body,
        out_type=jax.ShapeDtypeStruct((NC, G, D), _f32),
        mesh=_mesh(),
        compiler_params=pltpu.CompilerParams(needs_layout_passes=False),
        scratch_types=(
            pltpu.VMEM((CH,), _i32),
            pltpu.VMEM((CH, ED), _f32),
            pltpu.VMEM((CH, D), _f32),
            pltpu.VMEM((GSTRIPE, D), _f32),
            pltpu.VMEM_SHARED((G, D), _f32),
            pltpu.SemaphoreType.DMA,
        ),
    )
    return f(vals, seg)


def _hmm(a, b):
    return jnp.matmul(a, b, precision=jax.lax.Precision.HIGHEST)


def _graph_ln(x, oh, ohT, cnt, gamma, beta):
    mean = _hmm(oh, x.sum(-1)) / cnt
    var = _hmm(oh, (x * x).sum(-1)) / cnt - mean * mean
    rstd = jax.lax.rsqrt(jnp.maximum(var, 0.0) + 1e-5)
    xn = (x - _hmm(ohT, mean)[:, None]) * _hmm(ohT, rstd)[:, None]
    return xn * gamma + beta


def _batch_softmax(sc, oh, ohT, ohb):
    m = jnp.where(ohb, sc[None, :], -jnp.inf).max(axis=1)
    m = jnp.where(jnp.isfinite(m), m, 0.0)
    e = jnp.exp(sc - _hmm(ohT, m))
    d = _hmm(oh, e)
    return e / (_hmm(ohT, d) + 1e-16)


_USE_SC_CONV = True


def _conv_jax(x, src, dst, e0, e1, p):
    q = (x @ p['wq'] + p['bq'])[dst].reshape(-1, H, C)
    k = (x[src] @ p['wk'] + p['bk']).reshape(-1, H, C)
    e = jnp.concatenate([e0, e1], axis=1).reshape(-1, H, C)
    k = k + e
    alpha = (q * k).sum(-1) / jnp.sqrt(float(C))
    m = jax.ops.segment_max(alpha, dst, num_segments=N)
    m = jnp.where(jnp.isfinite(m), m, 0.0)
    ee = jnp.exp(alpha - m[dst])
    dd = jax.ops.segment_sum(ee, dst, num_segments=N)
    alpha = ee / (dd[dst] + 1e-16)
    v = (x[src] @ p['wv'] + p['bv']).reshape(-1, H, C) + e
    out = jax.ops.segment_sum(alpha[..., None] * v, dst, num_segments=N).reshape(N, H * C)
    return out + x @ p['wskip'] + p['bskip']


def _conv(x, src, dst, e0, e1, p):
    if not _USE_SC_CONV:
        return _conv_jax(x, src, dst, e0, e1, p)
    q = x @ p['wq'] + p['bq']
    kx = x @ p['wk'] + p['bk']
    vx = x @ p['wv'] + p['bv']
    lg0, lg1, mx = _k_logits(kx, q, e0, e1, src, dst)
    mv = jnp.full((LANES,), jnp.max(mx), _f32)
    acc = _k_scatter(vx, e0, e1, lg0, lg1, src, dst, mv)
    a0 = acc[0, 0] + acc[0, 1]
    a1 = acc[1, 0] + acc[1, 1]
    out = jnp.concatenate([a0[:, :C] / (a0[:, C:C + 1] + 1e-16),
                           a1[:, :C] / (a1[:, C:C + 1] + 1e-16)], axis=1)
    return out + x @ p['wskip'] + p['bskip']


def _sag_score(x, src, dst, oh, ohT, ohb, p):
    agg2 = _k_segsum(x, src, dst)
    agg = agg2[0] + agg2[1]
    sc = (agg @ p['wrel'] + p['brel'] + x @ p['wroot'])[:, 0]
    return _batch_softmax(sc, oh, ohT, ohb)


def _side_block(bp, x, ea, src, dst, eb, oh, ohT, ohb, cnt):
    w1, b1 = bp['conv1']['we'], bp['conv1']['be']
    x = _conv(x, src, dst, ea @ w1[:, :C] + b1[:C], ea @ w1[:, C:] + b1[C:],
              bp['conv1'])
    x = jax.nn.elu(_graph_ln(x, oh, ohT, cnt, bp['n1g'], bp['n1b']))
    ea = jax.nn.elu(ea @ bp['wu1'] + bp['bu1'])
    w2, b2 = bp['conv2']['we'], bp['conv2']['be']
    x = _conv(x, src, dst, ea @ w2[:, :C] + b2[:C], ea @ w2[:, C:] + b2[C:],
              bp['conv2'])
    ea = ea @ bp['wu2'] + bp['bu2']
    s = _sag_score(x, src, dst, oh, ohT, ohb, bp['pool'])
    g = _hmm(oh, x * s[:, None])
    eap = _k_edgepool(ea, eb)
    ge = jax.nn.elu((eap[0] + eap[1])[:, :ED] @ bp['wre'] + bp['bre'])
    g = g * ge
    x = jax.nn.elu(_graph_ln(x, oh, ohT, cnt, bp['n2g'], bp['n2b']))
    ea = jax.nn.elu(ea)
    return x, ea, g


def _l2n(v):
    return v / jnp.maximum(jnp.linalg.norm(v, axis=-1, keepdims=True), 1e-12)


_GB = 8


def _final_kernel(rh_ref, rt_ref, relm_ref, wk_ref, wq_ref, b_ref, a_ref, out_ref):
    rh = _l2n(rh_ref[...])
    rt = _l2n(rt_ref[...])
    keys = jax.lax.dot_general(rh, wk_ref[...], (((2,), (0,)), ((), ())))
    queries = jax.lax.dot_general(rt, wq_ref[...], (((2,), (0,)), ((), ())))
    eact = queries[:, None, :, :] + keys[:, :, None, :] + b_ref[...][0]
    att = jnp.sum(jnp.tanh(eact) * a_ref[...][0], axis=-1)
    relm = relm_ref[...]
    hr = jax.lax.dot_general(rh, relm, (((2,), (1,)), ((0,), (0,))))
    scores = jax.lax.dot_general(hr, rt, (((2,), (2,)), ((0,), (0,))))
    out_ref[...] = jnp.sum(att * scores, axis=(1, 2))[:, None]


def _final_stage(rh, rt, relm, wk_att, wq_att, b_att, a_att):
    return pl.pallas_call(
        _final_kernel,
        grid=(G // _GB,),
        in_specs=[
            pl.BlockSpec((_GB, NB, D), lambda i: (i, 0, 0)),
            pl.BlockSpec((_GB, NB, D), lambda i: (i, 0, 0)),
            pl.BlockSpec((_GB, D, D), lambda i: (i, 0, 0)),
            pl.BlockSpec((D, D // 2), lambda i: (0, 0)),
            pl.BlockSpec((D, D // 2), lambda i: (0, 0)),
            pl.BlockSpec((1, D // 2), lambda i: (0, 0)),
            pl.BlockSpec((1, D // 2), lambda i: (0, 0)),
        ],
        out_specs=pl.BlockSpec((_GB, 1), lambda i: (i, 0)),
        out_shape=jax.ShapeDtypeStruct((G, 1), jnp.float32),
    )(rh, rt, relm, wk_att, wq_att, b_att, a_att).reshape(G)


def kernel(params, h_x, h_edge_attr, t_x, t_edge_attr, h_fin, h_desc, t_fin, t_desc, h_edge_index, h_batch, h_edge_batch, t_edge_index, t_batch, t_edge_batch, rels):
    gids = jnp.arange(G, dtype=h_batch.dtype)
    h_ohb = h_batch[None, :] == gids[:, None]
    t_ohb = t_batch[None, :] == gids[:, None]
    h_oh = h_ohb.astype(_f32)
    t_oh = t_ohb.astype(_f32)
    h_ohT = h_oh.T
    t_ohT = t_oh.T
    h_cnt = jnp.maximum(h_oh.sum(1) * D, 1.0)
    t_cnt = jnp.maximum(t_oh.sum(1) * D, 1.0)

    hx = h_x @ params['w_in'] + params['b_in']
    tx = t_x @ params['w_in'] + params['b_in']
    hx = jax.nn.elu(_graph_ln(hx, h_oh, h_ohT, h_cnt, params['n0g'], params['n0b']))
    tx = jax.nn.elu(_graph_ln(tx, t_oh, t_ohT, t_cnt, params['n0g'], params['n0b']))
    hea = jax.nn.elu(h_edge_attr @ params['w_e'] + params['b_e'])
    tea = jax.nn.elu(t_edge_attr @ params['w_e'] + params['b_e'])

    hsrc, hdst = h_edge_index[0], h_edge_index[1]
    tsrc, tdst = t_edge_index[0], t_edge_index[1]

    reprs_h, reprs_t = [], []
    for bp in params['blocks']:
        hx, hea, hg = _side_block(bp, hx, hea, hsrc, hdst, h_edge_batch, h_oh, h_ohT, h_ohb, h_cnt)
        tx, tea, tg = _side_block(bp, tx, tea, tsrc, tdst, t_edge_batch, t_oh, t_ohT, t_ohb, t_cnt)
        reprs_h.append(hg)
        reprs_t.append(tg)

    hf = jax.nn.relu(h_fin @ params['w_fp'] + params['b_fp'])
    tf = jax.nn.relu(t_fin @ params['w_fp'] + params['b_fp'])
    rh = jnp.stack([_l2n(r + hf) for r in reprs_h], axis=1)
    rt = jnp.stack([_l2n(r + tf) for r in reprs_t], axis=1)
    relm = params['rel_emb'][rels].reshape(-1, D, D)
    return _final_stage(
        rh, rt, relm,
        params['wk_att'], params['wq_att'],
        params['b_att'].reshape(1, D // 2), params['a_att'].reshape(1, D // 2),
    )


# speedup vs baseline: 1.1666x; 1.0381x over previous
import functools

import jax
import jax.numpy as jnp
from jax import lax
from jax.experimental import pallas as pl
from jax.experimental.pallas import tpu as pltpu
from jax.experimental.pallas import tpu_sc as plsc

N = 10000
E = 320000
G = 256
NB = 2
H = 2
C = 64
D = 128
ED = 64

NC = 2
NS = 16
LANES = 16
NW = NC * NS
CH = 128
NBT = E // CH
NBLK = -(-NBT // NW)
NGRP = CH // 16
S0 = 624
GSTRIPE = G // NS

_f32 = jnp.float32
_i32 = jnp.int32


def _mesh():
    return plsc.VectorSubcoreMesh(core_axis_name="c", subcore_axis_name="s",
                                  num_cores=NC, num_subcores=NS)


def _zero_rows(ref, nrows, width):
    zero16 = jnp.zeros((LANES,), _f32)
    per = width // LANES

    def body(i, _):
        ref[i // per, pl.ds((i % per) * LANES, LANES)] = zero16
        return 0
    lax.fori_loop(0, nrows * per, body, 0)


def _k_logits_body(kx_hbm, q_hbm, e0_hbm, e1_hbm, src_hbm, dst_hbm,
                   lg0_out, lg1_out, mx_out,
                   srcv, dstv, kxg, qg, ev0, ev1, lv0, lv1, mxa, sem1, sem2):
    c = lax.axis_index("c")
    s = lax.axis_index("s")
    wid = s * NC + c
    iota = lax.iota(_i32, LANES)

    mxa[...] = jnp.full((LANES,), -1e30, _f32)

    def chunk(k, _):
        bid = k * NW + wid

        @pl.when(bid < NBT)
        def _():
            base = bid * CH
            ca = pltpu.async_copy(src_hbm.at[pl.ds(base, CH)], srcv, sem1)
            cb = pltpu.async_copy(dst_hbm.at[pl.ds(base, CH)], dstv, sem1)
            cc_ = pltpu.async_copy(e0_hbm.at[pl.ds(base, CH)], ev0, sem2)
            cd = pltpu.async_copy(e1_hbm.at[pl.ds(base, CH)], ev1, sem2)
            ca.wait()
            cb.wait()
            cp1 = pltpu.async_copy(kx_hbm.at[srcv], kxg, sem1)
            cp2 = pltpu.async_copy(q_hbm.at[dstv], qg, sem1)
            cc_.wait()
            cd.wait()
            cp1.wait()
            cp2.wait()

            def grp(g, _):
                rowv = g * 16 + iota

                def dot_head(ev, lo):
                    def cbody(c8, acc):
                        colv = jnp.zeros((LANES,), _i32) + c8 * 8 + lo
                        cole = jnp.zeros((LANES,), _i32) + c8 * 8
                        for u in range(8):
                            qv = plsc.load_gather(qg, [rowv, colv + u])
                            kv = (plsc.load_gather(kxg, [rowv, colv + u])
                                  + plsc.load_gather(ev, [rowv, cole + u]))
                            acc = acc + qv * kv
                        return acc
                    return lax.fori_loop(0, C // 8, cbody,
                                         jnp.zeros((LANES,), _f32))

                l0 = dot_head(ev0, 0) * 0.125
                l1 = dot_head(ev1, C) * 0.125
                lv0[pl.ds(g * 16, 16)] = l0
                lv1[pl.ds(g * 16, 16)] = l1
                mxa[...] = jnp.maximum(mxa[...], jnp.maximum(l0, l1))
                return 0

            lax.fori_loop(0, NGRP, grp, 0)
            cw1 = pltpu.async_copy(lv0, lg0_out.at[pl.ds(base, CH)], sem1)
            cw2 = pltpu.async_copy(lv1, lg1_out.at[pl.ds(base, CH)], sem1)
            cw1.wait()
            cw2.wait()
        return 0

    lax.fori_loop(0, NBLK, chunk, 0)
    lv0[pl.ds(0, 16)] = mxa[...]
    pltpu.sync_copy(lv0, mx_out.at[pl.ds(wid * CH, CH)])


def _k_logits(kx, q, e0, e1, src, dst):
    f = pl.kernel(
        _k_logits_body,
        out_type=(jax.ShapeDtypeStruct((E,), _f32),
                  jax.ShapeDtypeStruct((E,), _f32),
                  jax.ShapeDtypeStruct((NW * CH,), _f32)),
        mesh=_mesh(),
        compiler_params=pltpu.CompilerParams(needs_layout_passes=False),
        scratch_types=(
            pltpu.VMEM((CH,), _i32),
            pltpu.VMEM((CH,), _i32),
            pltpu.VMEM((CH, D), _f32),
            pltpu.VMEM((CH, D), _f32),
            pltpu.VMEM((CH, C), _f32),
            pltpu.VMEM((CH, C), _f32),
            pltpu.VMEM((CH,), _f32),
            pltpu.VMEM((CH,), _f32),
            pltpu.VMEM((LANES,), _f32),
            pltpu.SemaphoreType.DMA,
            pltpu.SemaphoreType.DMA,
        ),
    )
    return f(kx, q, e0, e1, src, dst)


def _zero_shared(zb, sh, s, width):
    def zrow(k, _):
        pltpu.sync_copy(zb, sh.at[pl.ds(s * S0 + k * 16, 16)])
        return 0
    lax.fori_loop(0, 39, zrow, 0)

    @pl.when(s == 0)
    def _():
        pltpu.sync_copy(zb, sh.at[pl.ds(NS * S0, 16)])


def _copy_out_shared(sh, out, c, s):
    pltpu.sync_copy(sh.at[pl.ds(s * S0, S0)], out.at[c].at[pl.ds(s * S0, S0)])

    @pl.when(s == 0)
    def _():
        pltpu.sync_copy(sh.at[pl.ds(NS * S0, 16)],
                        out.at[c].at[pl.ds(NS * S0, 16)])


def _k_scatter_body(vx_hbm, e0_hbm, e1_hbm, lg0_hbm, lg1_hbm,
                    src_hbm, dst_hbm, mv_hbm,
                    acc_out,
                    srcv, dstv, vxg, ev, rows, lv, mbuf,
                    acc_sh, sem1, sem2):
    c = lax.axis_index("c")
    s = lax.axis_index("s")
    wid = s * NC + c
    iota = lax.iota(_i32, LANES)

    pltpu.sync_copy(mv_hbm, mbuf)
    mvec = mbuf[...]

    for h, (e_hbm, lg_hbm) in enumerate(
            ((e0_hbm, lg0_hbm), (e1_hbm, lg1_hbm))):
        _zero_rows(rows, CH, D)
        plsc.subcore_barrier()
        _zero_shared(rows.at[pl.ds(0, 16)], acc_sh, s, D)
        plsc.subcore_barrier()

        def chunk(k, _):
            bid = k * NW + wid

            @pl.when(bid < NBT)
            def _():
                base = bid * CH
                ca = pltpu.async_copy(src_hbm.at[pl.ds(base, CH)], srcv, sem1)
                cb = pltpu.async_copy(dst_hbm.at[pl.ds(base, CH)], dstv, sem2)
                cc_ = pltpu.async_copy(e_hbm.at[pl.ds(base, CH)], ev, sem2)
                cd = pltpu.async_copy(lg_hbm.at[pl.ds(base, CH)], lv, sem2)
                ca.wait()
                cp1 = pltpu.async_copy(vx_hbm.at[srcv], vxg, sem1)
                cb.wait()
                cc_.wait()
                cd.wait()
                cp1.wait()

                def grp(g, _):
                    sh = jnp.exp(lv[pl.ds(g * 16, 16)] - mvec)
                    rowv = g * 16 + iota
                    plsc.store_scatter(
                        rows, [rowv, jnp.zeros((LANES,), _i32) + C], sh)

                    def cbody(c8, _):
                        colv = jnp.zeros((LANES,), _i32) + c8 * 8 + h * C
                        cole = jnp.zeros((LANES,), _i32) + c8 * 8
                        for u in range(8):
                            vv = (plsc.load_gather(vxg, [rowv, colv + u])
                                  + plsc.load_gather(ev, [rowv, cole + u]))
                            plsc.store_scatter(rows, [rowv, cole + u], sh * vv)
                        return 0
                    lax.fori_loop(0, C // 8, cbody, 0)
                    return 0

                lax.fori_loop(0, NGRP, grp, 0)
                pltpu.sync_copy(rows, acc_sh.at[dstv], add=True)
            return 0

        lax.fori_loop(0, NBLK, chunk, 0)
        plsc.subcore_barrier()
        _copy_out_shared(acc_sh, acc_out.at[h], c, s)


def _k_scatter(vx, e0, e1, lg0, lg1, src, dst, mv):
    f = pl.kernel(
        _k_scatter_body,
        out_type=jax.ShapeDtypeStruct((H, NC, N, D), _f32),
        mesh=_mesh(),
        compiler_params=pltpu.CompilerParams(needs_layout_passes=False),
        scratch_types=(
            pltpu.VMEM((CH,), _i32),
            pltpu.VMEM((CH,), _i32),
            pltpu.VMEM((CH, D), _f32),
            pltpu.VMEM((CH, C), _f32),
            pltpu.VMEM((CH, D), _f32),
            pltpu.VMEM((CH,), _f32),
            pltpu.VMEM((LANES,), _f32),
            pltpu.VMEM_SHARED((N, D), _f32),
            pltpu.SemaphoreType.DMA,
            pltpu.SemaphoreType.DMA,
        ),
    )
    return f(vx, e0, e1, lg0, lg1, src, dst, mv)


def _k_segsum_body(x_hbm, src_hbm, dst_hbm, acc_out,
                   srcv, dstv, xg, zb, acc_sh, sem1):
    c = lax.axis_index("c")
    s = lax.axis_index("s")
    wid = s * NC + c

    _zero_rows(zb, 16, D)
    _zero_shared(zb, acc_sh, s, D)
    plsc.subcore_barrier()

    def chunk(k, _):
        bid = k * NW + wid

        @pl.when(bid < NBT)
        def _():
            base = bid * CH
            ca = pltpu.async_copy(src_hbm.at[pl.ds(base, CH)], srcv, sem1)
            cb = pltpu.async_copy(dst_hbm.at[pl.ds(base, CH)], dstv, sem1)
            ca.wait()
            cb.wait()
            pltpu.async_copy(x_hbm.at[srcv], xg, sem1).wait()
            pltpu.sync_copy(xg, acc_sh.at[dstv], add=True)
        return 0

    lax.fori_loop(0, NBLK, chunk, 0)
    plsc.subcore_barrier()
    _copy_out_shared(acc_sh, acc_out, c, s)


def _k_segsum(x, src, dst):
    f = pl.kernel(
        _k_segsum_body,
        out_type=jax.ShapeDtypeStruct((NC, N, D), _f32),
        mesh=_mesh(),
        compiler_params=pltpu.CompilerParams(needs_layout_passes=False),
        scratch_types=(
            pltpu.VMEM((CH,), _i32),
            pltpu.VMEM((CH,), _i32),
            pltpu.VMEM((CH, D), _f32),
            pltpu.VMEM((16, D), _f32),
            pltpu.VMEM_SHARED((N, D), _f32),
            pltpu.SemaphoreType.DMA,
        ),
    )
    return f(x, src, dst)


def _k_edgepool_body(v_hbm, seg_hbm, acc_out, segv, vv, vp, zb, acc_sh, sem1):
    c = lax.axis_index("c")
    s = lax.axis_index("s")
    wid = s * NC + c

    _zero_rows(vp, CH, D)
    _zero_rows(zb, GSTRIPE, D)
    pltpu.sync_copy(zb, acc_sh.at[pl.ds(s * GSTRIPE, GSTRIPE)])
    plsc.subcore_barrier()

    def chunk(k, _):
        bid = k * NW + wid

        @pl.when(bid < NBT)
        def _():
            base = bid * CH
            ca = pltpu.async_copy(seg_hbm.at[pl.ds(base, CH)], segv, sem1)
            cb = pltpu.async_copy(v_hbm.at[pl.ds(base, CH)], vv, sem1)
            ca.wait()
            cb.wait()

            def cprow(i, _):
                for r in range(ED // LANES):
                    vp[i, pl.ds(r * 16, 16)] = vv[i, pl.ds(r * 16, 16)]
                return 0
            lax.fori_loop(0, CH, cprow, 0)
            pltpu.sync_copy(vp, acc_sh.at[segv], add=True)
        return 0

    lax.fori_loop(0, NBLK, chunk, 0)
    plsc.subcore_barrier()
    off = s * GSTRIPE
    pltpu.sync_copy(acc_sh.at[pl.ds(off, GSTRIPE)],
                    acc_out.at[c].at[pl.ds(off, GSTRIPE)])


def _k_edgepool(vals, seg):
    f = pl.kernel(
        _k_edgepool_body,
        out_type=jax.ShapeDtypeStruct((NC, G, D), _f32),
        mesh=_mesh(),
        compiler_params=pltpu.CompilerParams(needs_layout_passes=False),
        scratch_types=(
            pltpu.VMEM((CH,), _i32),
            pltpu.VMEM((CH, ED), _f32),
            pltpu.VMEM((CH, D), _f32),
            pltpu.VMEM((GSTRIPE, D), _f32),
            pltpu.VMEM_SHARED((G, D), _f32),
            pltpu.SemaphoreType.DMA,
        ),
    )
    return f(vals, seg)


def _hmm(a, b):
    return jnp.matmul(a, b, precision=jax.lax.Precision.HIGHEST)


def _graph_ln(x, oh, ohT, cnt, gamma, beta):
    mean = _hmm(oh, x.sum(-1)) / cnt
    var = _hmm(oh, (x * x).sum(-1)) / cnt - mean * mean
    rstd = jax.lax.rsqrt(jnp.maximum(var, 0.0) + 1e-5)
    xn = (x - _hmm(ohT, mean)[:, None]) * _hmm(ohT, rstd)[:, None]
    return xn * gamma + beta


def _batch_softmax(sc, oh, ohT, ohb):
    m = jnp.where(ohb, sc[None, :], -jnp.inf).max(axis=1)
    m = jnp.where(jnp.isfinite(m), m, 0.0)
    e = jnp.exp(sc - _hmm(ohT, m))
    d = _hmm(oh, e)
    return e / (_hmm(ohT, d) + 1e-16)


_USE_SC_CONV = True


def _conv_jax(x, src, dst, e0, e1, p):
    q = (x @ p['wq'] + p['bq'])[dst].reshape(-1, H, C)
    k = (x[src] @ p['wk'] + p['bk']).reshape(-1, H, C)
    e = jnp.concatenate([e0, e1], axis=1).reshape(-1, H, C)
    k = k + e
    alpha = (q * k).sum(-1) / jnp.sqrt(float(C))
    m = jax.ops.segment_max(alpha, dst, num_segments=N)
    m = jnp.where(jnp.isfinite(m), m, 0.0)
    ee = jnp.exp(alpha - m[dst])
    dd = jax.ops.segment_sum(ee, dst, num_segments=N)
    alpha = ee / (dd[dst] + 1e-16)
    v = (x[src] @ p['wv'] + p['bv']).reshape(-1, H, C) + e
    out = jax.ops.segment_sum(alpha[..., None] * v, dst, num_segments=N).reshape(N, H * C)
    return out + x @ p['wskip'] + p['bskip']


def _conv(x, src, dst, e0, e1, p):
    if not _USE_SC_CONV:
        return _conv_jax(x, src, dst, e0, e1, p)
    q = x @ p['wq'] + p['bq']
    kx = x @ p['wk'] + p['bk']
    vx = x @ p['wv'] + p['bv']
    lg0, lg1, mx = _k_logits(kx, q, e0, e1, src, dst)
    mv = jnp.full((LANES,), jnp.max(mx), _f32)
    acc = _k_scatter(vx, e0, e1, lg0, lg1, src, dst, mv)
    a0 = acc[0, 0] + acc[0, 1]
    a1 = acc[1, 0] + acc[1, 1]
    out = jnp.concatenate([a0[:, :C] / (a0[:, C:C + 1] + 1e-16),
                           a1[:, :C] / (a1[:, C:C + 1] + 1e-16)], axis=1)
    return out + x @ p['wskip'] + p['bskip']


def _sag_score(x, src, dst, oh, ohT, ohb, p):
    agg2 = _k_segsum(x, src, dst)
    agg = agg2[0] + agg2[1]
    sc = (agg @ p['wrel'] + p['brel'] + x @ p['wroot'])[:, 0]
    return _batch_softmax(sc, oh, ohT, ohb)


def _side_block(bp, x, ea, src, dst, eb, oh, ohT, ohb, cnt):
    w1, b1 = bp['conv1']['we'], bp['conv1']['be']
    x = _conv(x, src, dst, ea @ w1[:, :C] + b1[:C], ea @ w1[:, C:] + b1[C:],
              bp['conv1'])
    x = jax.nn.elu(_graph_ln(x, oh, ohT, cnt, bp['n1g'], bp['n1b']))
    ea = jax.nn.elu(ea @ bp['wu1'] + bp['bu1'])
    w2, b2 = bp['conv2']['we'], bp['conv2']['be']
    x = _conv(x, src, dst, ea @ w2[:, :C] + b2[:C], ea @ w2[:, C:] + b2[C:],
              bp['conv2'])
    ea = ea @ bp['wu2'] + bp['bu2']
    s = _sag_score(x, src, dst, oh, ohT, ohb, bp['pool'])
    g = _hmm(oh, x * s[:, None])
    eap = _k_edgepool(ea, eb)
    ge = jax.nn.elu((eap[0] + eap[1])[:, :ED] @ bp['wre'] + bp['bre'])
    g = g * ge
    x = jax.nn.elu(_graph_ln(x, oh, ohT, cnt, bp['n2g'], bp['n2b']))
    ea = jax.nn.elu(ea)
    return x, ea, g


def _l2n(v):
    return v / jnp.maximum(jnp.linalg.norm(v, axis=-1, keepdims=True), 1e-12)


_GB = 8


def _final_kernel(rh_ref, rt_ref, relm_ref, wk_ref, wq_ref, b_ref, a_ref, out_ref):
    rh = _l2n(rh_ref[...])
    rt = _l2n(rt_ref[...])
    keys = jax.lax.dot_general(rh, wk_ref[...], (((2,), (0,)), ((), ())))
    queries = jax.lax.dot_general(rt, wq_ref[...], (((2,), (0,)), ((), ())))
    eact = queries[:, None, :, :] + keys[:, :, None, :] + b_ref[...][0]
    att = jnp.sum(jnp.tanh(eact) * a_ref[...][0], axis=-1)
    relm = relm_ref[...]
    hr = jax.lax.dot_general(rh, relm, (((2,), (1,)), ((0,), (0,))))
    scores = jax.lax.dot_general(hr, rt, (((2,), (2,)), ((0,), (0,))))
    out_ref[...] = jnp.sum(att * scores, axis=(1, 2))[:, None]


def _final_stage(rh, rt, relm, wk_att, wq_att, b_att, a_att):
    return pl.pallas_call(
        _final_kernel,
        grid=(G // _GB,),
        in_specs=[
            pl.BlockSpec((_GB, NB, D), lambda i: (i, 0, 0)),
            pl.BlockSpec((_GB, NB, D), lambda i: (i, 0, 0)),
            pl.BlockSpec((_GB, D, D), lambda i: (i, 0, 0)),
            pl.BlockSpec((D, D // 2), lambda i: (0, 0)),
            pl.BlockSpec((D, D // 2), lambda i: (0, 0)),
            pl.BlockSpec((1, D // 2), lambda i: (0, 0)),
            pl.BlockSpec((1, D // 2), lambda i: (0, 0)),
        ],
        out_specs=pl.BlockSpec((_GB, 1), lambda i: (i, 0)),
        out_shape=jax.ShapeDtypeStruct((G, 1), jnp.float32),
    )(rh, rt, relm, wk_att, wq_att, b_att, a_att).reshape(G)


def kernel(params, h_x, h_edge_attr, t_x, t_edge_attr, h_fin, h_desc, t_fin, t_desc, h_edge_index, h_batch, h_edge_batch, t_edge_index, t_batch, t_edge_batch, rels):
    gids = jnp.arange(G, dtype=h_batch.dtype)
    h_ohb = h_batch[None, :] == gids[:, None]
    t_ohb = t_batch[None, :] == gids[:, None]
    h_oh = h_ohb.astype(_f32)
    t_oh = t_ohb.astype(_f32)
    h_ohT = h_oh.T
    t_ohT = t_oh.T
    h_cnt = jnp.maximum(h_oh.sum(1) * D, 1.0)
    t_cnt = jnp.maximum(t_oh.sum(1) * D, 1.0)

    hx = h_x @ params['w_in'] + params['b_in']
    tx = t_x @ params['w_in'] + params['b_in']
    hx = jax.nn.elu(_graph_ln(hx, h_oh, h_ohT, h_cnt, params['n0g'], params['n0b']))
    tx = jax.nn.elu(_graph_ln(tx, t_oh, t_ohT, t_cnt, params['n0g'], params['n0b']))
    hea = jax.nn.elu(h_edge_attr @ params['w_e'] + params['b_e'])
    tea = jax.nn.elu(t_edge_attr @ params['w_e'] + params['b_e'])

    hsrc, hdst = h_edge_index[0], h_edge_index[1]
    tsrc, tdst = t_edge_index[0], t_edge_index[1]

    reprs_h, reprs_t = [], []
    for bp in params['blocks']:
        hx, hea, hg = _side_block(bp, hx, hea, hsrc, hdst, h_edge_batch, h_oh, h_ohT, h_ohb, h_cnt)
        tx, tea, tg = _side_block(bp, tx, tea, tsrc, tdst, t_edge_batch, t_oh, t_ohT, t_ohb, t_cnt)
        reprs_h.append(hg)
        reprs_t.append(tg)

    hf = jax.nn.relu(h_fin @ params['w_fp'] + params['b_fp'])
    tf = jax.nn.relu(t_fin @ params['w_fp'] + params['b_fp'])
    rh = jnp.stack([_l2n(r + hf) for r in reprs_h], axis=1)
    rt = jnp.stack([_l2n(r + tf) for r in reprs_t], axis=1)
    relm = params['rel_emb'][rels].reshape(-1, D, D)
    return _final_stage(
        rh, rt, relm,
        params['wk_att'], params['wq_att'],
        params['b_att'].reshape(1, D // 2), params['a_att'].reshape(1, D // 2),
    )
